# baseline (speedup 1.0000x reference)
import math
import functools

import jax
import jax.numpy as jnp
from jax.experimental import pallas as pl
from jax.experimental.pallas import tpu as pltpu

B, L = 1, 2048
D_MODEL, D_KEY, D_QUERY, N_HEADS = 768, 64, 64, 8
RANK, RFF_DIM, K_MAX = 32, 64, 8
LSH_BUCKETS, LSH_BANDWIDTH, LSH_KEY_DIM, N_LSH_HASHES = 8, 16.0, 64, 2
NB = LSH_BUCKETS ** N_LSH_HASHES
NC = NB * K_MAX
RFF_SCALE = math.sqrt(2.0 / RFF_DIM)
SIM_SCALE = math.sqrt(RFF_DIM)
CHUNK = 256
N_CORES = 2
HEADS_PER_CORE = N_HEADS // N_CORES


def _hash_digits(x, lshv):
    xb = (jax.lax.bitcast_convert_type(x, jnp.int32) > 0).astype(jnp.float32)
    proj = jnp.dot(xb, lshv, preferred_element_type=jnp.float32)
    return jnp.floor(proj / LSH_BANDWIDTH) % LSH_BUCKETS


def _combine(width, scale=1.0):
    r = jax.lax.broadcasted_iota(jnp.int32, (N_LSH_HASHES, width), 0)
    return jnp.where(r == 0, scale * LSH_BUCKETS, scale).astype(jnp.float32)


def _head_kernel(xq_ref, xk_ref, xv_ref, wq_ref, bq_ref, wk_ref, bk_ref,
                 wv_ref, bv_ref, omega_ref, rffb_ref, lsh_ref, uv_ref,
                 vv_ref, wo_ref, bo_ref, o_ref):
    cid = pl.program_id(0)
    hid = pl.program_id(1)
    qh = jnp.dot(xq_ref[...], wq_ref[0],
                 preferred_element_type=jnp.float32) + bq_ref[0]
    kh = jnp.dot(xk_ref[...], wk_ref[0],
                 preferred_element_type=jnp.float32) + bk_ref[0]
    vh = jnp.dot(xv_ref[...], wv_ref[0],
                 preferred_element_type=jnp.float32) + bv_ref[0]
    omega = omega_ref[0]
    rffb = rffb_ref[0]
    lshv = lsh_ref[0]

    hq = _hash_digits(qh, lshv)
    hk = _hash_digits(kh, lshv)
    cq_bc = jnp.dot(hq, _combine(NC),
                    preferred_element_type=jnp.float32)
    ck_bc64 = jnp.dot(hk, _combine(NB),
                      preferred_element_type=jnp.float32)
    ck8_bc = jnp.dot(hk, _combine(NC, scale=float(K_MAX)),
                     preferred_element_type=jnp.float32)

    iota64 = jax.lax.broadcasted_iota(jnp.int32, (L, NB), 1)
    onehot = (ck_bc64.astype(jnp.int32) == iota64).astype(jnp.float32)
    tril = (jax.lax.broadcasted_iota(jnp.int32, (CHUNK, CHUNK), 0)
            >= jax.lax.broadcasted_iota(jnp.int32, (CHUNK, CHUNK), 1)
            ).astype(jnp.float32)
    counts = jnp.zeros((1, NB), jnp.float32)
    ranks = []
    for c in range(L // CHUNK):
        blk = onehot[c * CHUNK:(c + 1) * CHUNK]
        csum = jnp.dot(tril, blk, preferred_element_type=jnp.float32) + counts
        ranks.append(jnp.sum(csum * blk, axis=1, keepdims=True))
        counts = counts + jnp.sum(blk, axis=0, keepdims=True)
    rank = jnp.concatenate(ranks, axis=0)

    slotfix = jnp.where(rank <= K_MAX, rank - 1.0, 4.0 * NC)
    ones_row = jnp.full((1, NC), 1.0, jnp.float32)
    slot_bc = ck8_bc + jnp.dot(slotfix, ones_row,
                               preferred_element_type=jnp.float32)
    col = jax.lax.broadcasted_iota(jnp.int32, (L, NC), 1)
    sel = (slot_bc.astype(jnp.int32) == col).astype(jnp.float32)

    kr = jnp.cos(jnp.dot(kh, omega, preferred_element_type=jnp.float32)
                 + rffb)
    kr_tt = jax.lax.dot_general(kr, sel, (((0,), (0,)), ((), ())),
                                preferred_element_type=jnp.float32)
    cv_tt = jax.lax.dot_general(vh, sel, (((0,), (0,)), ((), ())),
                                preferred_element_type=jnp.float32)

    expand = (jax.lax.broadcasted_iota(jnp.int32, (NB, NC), 0)
              == jax.lax.broadcasted_iota(jnp.int32, (NB, NC), 1) // K_MAX
              ).astype(jnp.float32)
    cnt_col = jnp.dot(counts, expand, preferred_element_type=jnp.float32)
    col_row = jax.lax.broadcasted_iota(jnp.int32, (1, NC), 1)
    ctab = jnp.where(cnt_col > (col_row % K_MAX).astype(jnp.float32),
                     col_row >> 3, -1)

    qr = jnp.cos(jnp.dot(qh, omega, preferred_element_type=jnp.float32)
                 + rffb) * (RFF_SCALE * RFF_SCALE / SIM_SCALE)
    sim = jnp.dot(qr, kr_tt, preferred_element_type=jnp.float32)
    simm = jnp.where(cq_bc.astype(jnp.int32) == ctab, sim, -jnp.inf)
    m = jnp.max(simm, axis=1, keepdims=True)
    p = jnp.exp(simm - m)
    wvu = jax.lax.dot_general(p, cv_tt, (((1,), (1,)), ((), ())),
                              preferred_element_type=jnp.float32)
    wv = wvu * (1.0 / jnp.sum(p, axis=1, keepdims=True))
    r = jnp.dot(wv, uv_ref[0], preferred_element_type=jnp.float32)

    n = jnp.dot(vv_ref[0], wo_ref[...], preferred_element_type=jnp.float32)
    acc = jnp.dot(r, n, preferred_element_type=jnp.float32)

    @pl.when(jnp.logical_and(hid == 0, cid == 0))
    def _():
        o_ref[0] = acc + bo_ref[...]

    @pl.when(jnp.logical_and(hid == 0, cid != 0))
    def _():
        o_ref[0] = acc

    @pl.when(hid != 0)
    def _():
        o_ref[0] += acc


@functools.partial(jax.jit, static_argnames=("interpret",))
def kernel(query, key, value, Wq, bq, Wk, bk, Wv, bv, Uv, Vv, omega,
           rff_bias, lsh_vecs, Wo, bo, interpret=False):
    x_q = query.reshape(L, D_MODEL)
    x_k = key.reshape(L, D_MODEL)
    x_v = value.reshape(L, D_MODEL)
    wq_t = Wq.reshape(D_MODEL, N_HEADS, D_QUERY).transpose(1, 0, 2)
    wk_t = Wk.reshape(D_MODEL, N_HEADS, D_KEY).transpose(1, 0, 2)
    wv_t = Wv.reshape(D_MODEL, N_HEADS, D_KEY).transpose(1, 0, 2)
    bq3 = bq.reshape(N_HEADS, 1, D_QUERY)
    bk3 = bk.reshape(N_HEADS, 1, D_KEY)
    bv3 = bv.reshape(N_HEADS, 1, D_KEY)
    bo2 = bo.reshape(1, -1)
    rffb3 = rff_bias.reshape(N_HEADS, 1, RFF_DIM)

    def head_spec(d2, d3):
        return pl.BlockSpec(
            (1, d2, d3), lambda c, h: (c * HEADS_PER_CORE + h, 0, 0))

    partial = pl.pallas_call(
        _head_kernel,
        grid=(N_CORES, HEADS_PER_CORE),
        in_specs=[
            pl.BlockSpec((L, D_MODEL), lambda c, h: (0, 0)),
            pl.BlockSpec((L, D_MODEL), lambda c, h: (0, 0)),
            pl.BlockSpec((L, D_MODEL), lambda c, h: (0, 0)),
            head_spec(D_MODEL, D_QUERY),
            head_spec(1, D_QUERY),
            head_spec(D_MODEL, D_KEY),
            head_spec(1, D_KEY),
            head_spec(D_MODEL, D_KEY),
            head_spec(1, D_KEY),
            head_spec(D_KEY, RFF_DIM),
            head_spec(1, RFF_DIM),
            head_spec(LSH_KEY_DIM, N_LSH_HASHES),
            head_spec(D_KEY, RANK),
            head_spec(RANK, D_MODEL),
            pl.BlockSpec((D_MODEL, D_MODEL),
                         lambda c, h: (c * HEADS_PER_CORE + h, 0)),
            pl.BlockSpec((1, D_MODEL), lambda c, h: (0, 0)),
        ],
        out_specs=pl.BlockSpec((1, L, D_MODEL), lambda c, h: (c, 0, 0)),
        out_shape=jax.ShapeDtypeStruct((N_CORES, L, D_MODEL), jnp.float32),
        compiler_params=pltpu.CompilerParams(
            dimension_semantics=("parallel", "arbitrary"),
            fuse_transposed_lhs_in_matmul=True),
        interpret=interpret,
    )(x_q, x_k, x_v, wq_t, bq3, wk_t, bk3, wv_t, bv3, omega, rffb3,
      lsh_vecs, Uv, Vv, Wo, bo2)

    out = partial[0] + partial[1]
    return out.reshape(B, L, D_MODEL)


# speedup vs baseline: 1.0485x; 1.0485x over previous
import math
import functools

import jax
import jax.numpy as jnp
from jax.experimental import pallas as pl
from jax.experimental.pallas import tpu as pltpu

B, L = 1, 2048
D_MODEL, D_KEY, D_QUERY, N_HEADS = 768, 64, 64, 8
RANK, RFF_DIM, K_MAX = 32, 64, 8
LSH_BUCKETS, LSH_BANDWIDTH, LSH_KEY_DIM, N_LSH_HASHES = 8, 16.0, 64, 2
NB = LSH_BUCKETS ** N_LSH_HASHES
NC = NB * K_MAX
RFF_SCALE = math.sqrt(2.0 / RFF_DIM)
SIM_SCALE = math.sqrt(RFF_DIM)
CHUNK = 256
N_CORES = 2
HEADS_PER_CORE = N_HEADS // N_CORES


def _hash_digits(x, lshv):
    xb = (jax.lax.bitcast_convert_type(x, jnp.int32) > 0).astype(jnp.float32)
    proj = jnp.dot(xb, lshv, preferred_element_type=jnp.float32)
    return jnp.floor(proj / LSH_BANDWIDTH) % LSH_BUCKETS


def _combine(width, scale=1.0):
    r = jax.lax.broadcasted_iota(jnp.int32, (N_LSH_HASHES, width), 0)
    return jnp.where(r == 0, scale * LSH_BUCKETS, scale).astype(jnp.float32)


def _head_kernel(xq_ref, xk_ref, xv_ref, wq_ref, bq_ref, wk_ref, bk_ref,
                 wv_ref, bv_ref, omega_ref, rffb_ref, lsh_ref, uv_ref,
                 vv_ref, wo_ref, bo_ref, o_ref):
    hid = pl.program_id(0)
    qh = jnp.dot(xq_ref[...], wq_ref[0],
                 preferred_element_type=jnp.float32) + bq_ref[0]
    kh = jnp.dot(xk_ref[...], wk_ref[0],
                 preferred_element_type=jnp.float32) + bk_ref[0]
    vh = jnp.dot(xv_ref[...], wv_ref[0],
                 preferred_element_type=jnp.float32) + bv_ref[0]
    omega = omega_ref[0]
    rffb = rffb_ref[0]
    lshv = lsh_ref[0]

    hq = _hash_digits(qh, lshv)
    hk = _hash_digits(kh, lshv)
    cq_bc = jnp.dot(hq, _combine(NC),
                    preferred_element_type=jnp.float32)
    ck_bc64 = jnp.dot(hk, _combine(NB),
                      preferred_element_type=jnp.float32)
    ck8_bc = jnp.dot(hk, _combine(NC, scale=float(K_MAX)),
                     preferred_element_type=jnp.float32)

    iota64 = jax.lax.broadcasted_iota(jnp.int32, (L, NB), 1)
    onehot = (ck_bc64.astype(jnp.int32) == iota64).astype(jnp.float32)
    tril = (jax.lax.broadcasted_iota(jnp.int32, (CHUNK, CHUNK), 0)
            >= jax.lax.broadcasted_iota(jnp.int32, (CHUNK, CHUNK), 1)
            ).astype(jnp.float32)
    counts = jnp.zeros((1, NB), jnp.float32)
    ranks = []
    for c in range(L // CHUNK):
        blk = onehot[c * CHUNK:(c + 1) * CHUNK]
        csum = jnp.dot(tril, blk, preferred_element_type=jnp.float32) + counts
        ranks.append(jnp.sum(csum * blk, axis=1, keepdims=True))
        counts = counts + jnp.sum(blk, axis=0, keepdims=True)
    rank = jnp.concatenate(ranks, axis=0)

    slotfix = jnp.where(rank <= K_MAX, rank - 1.0, 4.0 * NC)
    ones_row = jnp.full((1, NC), 1.0, jnp.float32)
    slot_bc = ck8_bc + jnp.dot(slotfix, ones_row,
                               preferred_element_type=jnp.float32)
    col = jax.lax.broadcasted_iota(jnp.int32, (L, NC), 1)
    sel = (slot_bc.astype(jnp.int32) == col).astype(jnp.float32)

    kr = jnp.cos(jnp.dot(kh, omega, preferred_element_type=jnp.float32)
                 + rffb)
    kr_tt = jax.lax.dot_general(kr, sel, (((0,), (0,)), ((), ())),
                                preferred_element_type=jnp.float32)
    cv_tt = jax.lax.dot_general(vh, sel, (((0,), (0,)), ((), ())),
                                preferred_element_type=jnp.float32)

    expand = (jax.lax.broadcasted_iota(jnp.int32, (NB, NC), 0)
              == jax.lax.broadcasted_iota(jnp.int32, (NB, NC), 1) // K_MAX
              ).astype(jnp.float32)
    cnt_col = jnp.dot(counts, expand, preferred_element_type=jnp.float32)
    col_row = jax.lax.broadcasted_iota(jnp.int32, (1, NC), 1)
    ctab = jnp.where(cnt_col > (col_row % K_MAX).astype(jnp.float32),
                     col_row >> 3, -1)

    qr = jnp.cos(jnp.dot(qh, omega, preferred_element_type=jnp.float32)
                 + rffb) * (RFF_SCALE * RFF_SCALE / SIM_SCALE)
    sim = jnp.dot(qr, kr_tt, preferred_element_type=jnp.float32)
    simm = jnp.where(cq_bc.astype(jnp.int32) == ctab, sim, -jnp.inf)
    m = jnp.max(simm, axis=1, keepdims=True)
    p = jnp.exp(simm - m)
    wvu = jax.lax.dot_general(p, cv_tt, (((1,), (1,)), ((), ())),
                              preferred_element_type=jnp.float32)
    wv = wvu * (1.0 / jnp.sum(p, axis=1, keepdims=True))
    r = jnp.dot(wv, uv_ref[0], preferred_element_type=jnp.float32)

    n = jnp.dot(vv_ref[0], wo_ref[...], preferred_element_type=jnp.float32)
    acc = jnp.dot(r, n, preferred_element_type=jnp.float32)

    @pl.when(hid == 0)
    def _():
        o_ref[...] = acc + bo_ref[...]

    @pl.when(hid != 0)
    def _():
        o_ref[...] += acc


@functools.partial(jax.jit, static_argnames=("interpret",))
def kernel(query, key, value, Wq, bq, Wk, bk, Wv, bv, Uv, Vv, omega,
           rff_bias, lsh_vecs, Wo, bo, interpret=False):
    x_q = query.reshape(L, D_MODEL)
    x_k = key.reshape(L, D_MODEL)
    x_v = value.reshape(L, D_MODEL)
    wq_t = Wq.reshape(D_MODEL, N_HEADS, D_QUERY).transpose(1, 0, 2)
    wk_t = Wk.reshape(D_MODEL, N_HEADS, D_KEY).transpose(1, 0, 2)
    wv_t = Wv.reshape(D_MODEL, N_HEADS, D_KEY).transpose(1, 0, 2)
    bq3 = bq.reshape(N_HEADS, 1, D_QUERY)
    bk3 = bk.reshape(N_HEADS, 1, D_KEY)
    bv3 = bv.reshape(N_HEADS, 1, D_KEY)
    bo2 = bo.reshape(1, -1)
    rffb3 = rff_bias.reshape(N_HEADS, 1, RFF_DIM)

    def head_spec(d2, d3):
        return pl.BlockSpec((1, d2, d3), lambda h: (h, 0, 0))

    out = pl.pallas_call(
        _head_kernel,
        grid=(N_HEADS,),
        in_specs=[
            pl.BlockSpec((L, D_MODEL), lambda h: (0, 0)),
            pl.BlockSpec((L, D_MODEL), lambda h: (0, 0)),
            pl.BlockSpec((L, D_MODEL), lambda h: (0, 0)),
            head_spec(D_MODEL, D_QUERY),
            head_spec(1, D_QUERY),
            head_spec(D_MODEL, D_KEY),
            head_spec(1, D_KEY),
            head_spec(D_MODEL, D_KEY),
            head_spec(1, D_KEY),
            head_spec(D_KEY, RFF_DIM),
            head_spec(1, RFF_DIM),
            head_spec(LSH_KEY_DIM, N_LSH_HASHES),
            head_spec(D_KEY, RANK),
            head_spec(RANK, D_MODEL),
            pl.BlockSpec((D_MODEL, D_MODEL), lambda h: (h, 0)),
            pl.BlockSpec((1, D_MODEL), lambda h: (0, 0)),
        ],
        out_specs=pl.BlockSpec((L, D_MODEL), lambda h: (0, 0)),
        out_shape=jax.ShapeDtypeStruct((L, D_MODEL), jnp.float32),
        compiler_params=pltpu.CompilerParams(
            fuse_transposed_lhs_in_matmul=True),
        interpret=interpret,
    )(x_q, x_k, x_v, wq_t, bq3, wk_t, bk3, wv_t, bv3, omega, rffb3,
      lsh_vecs, Uv, Vv, Wo, bo2)

    return out.reshape(B, L, D_MODEL)
